# baseline (speedup 1.0000x reference)
import functools

import jax
import jax.numpy as jnp
from jax import lax
from jax.experimental import pallas as pl
from jax.experimental.pallas import tpu as pltpu
from jax.experimental.pallas import tpu_sc as plsc

_K = 128
_W = 512
_L = 16


def _cexe(ref, base, c0, p, dexp, k):
    dd = 1 << dexp
    r1 = base + (((p >> dexp) << (dexp + 1)) | (p & (dd - 1)))
    a = ref[r1, pl.ds(c0, _L)]
    b = ref[r1 + dd, pl.ds(c0, _L)]
    mx = jnp.maximum(a, b)
    mn = jnp.minimum(a, b)
    if k >= 7:
        ref[r1, pl.ds(c0, _L)] = mx
        ref[r1 + dd, pl.ds(c0, _L)] = mn
    else:
        desc = (((r1 & 127) >> k) & 1) == 0
        ref[r1, pl.ds(c0, _L)] = jnp.where(desc, mx, mn)
        ref[r1 + dd, pl.ds(c0, _L)] = jnp.where(desc, mn, mx)


def _clean128(ref, base, c0):
    for dexp in range(6, -1, -1):
        def cl(p, c, dexp=dexp):
            _cexe(ref, base, c0, p, dexp, 7)
            return c

        lax.fori_loop(0, 64, cl, 0, unroll=4)


def _merge_128(ref, a_base, b_base, o_base, c0):
    def hc(i, c):
        m = jnp.maximum(ref[a_base + i, pl.ds(c0, _L)],
                        ref[b_base + 127 - i, pl.ds(c0, _L)])
        ref[o_base + i, pl.ds(c0, _L)] = m
        return c

    lax.fori_loop(0, 128, hc, 0, unroll=4)
    _clean128(ref, o_base, c0)


def kernel(inputs):
    b, s, d = inputs.shape
    x2d = inputs.reshape(b * s, d)
    nwin = s // _W
    ntiles = b * (d // 128)
    mesh = plsc.VectorSubcoreMesh(core_axis_name="c", subcore_axis_name="s")

    @functools.partial(
        pl.kernel,
        out_type=jax.ShapeDtypeStruct((b * _K, d), jnp.float32),
        mesh=mesh,
        scratch_types=[
            pltpu.VMEM((_W, 128), jnp.float32),
            pltpu.VMEM((_K, 128), jnp.float32),
        ],
    )
    def sc(x_hbm, o_hbm, win, acc):
        wid = lax.axis_index("s") * 2 + lax.axis_index("c")
        per_worker = ntiles // 32

        def per_tile(t, carry):
            g = wid * per_worker + t
            bb = g // (d // 128)
            col0 = (g % (d // 128)) * 128

            def ini(i, c):
                r = i >> 3
                cc = pl.multiple_of((i & 7) * _L, _L)
                acc[r, pl.ds(cc, _L)] = jnp.full((_L,), -jnp.inf, jnp.float32)
                return c

            lax.fori_loop(0, _K * 8, ini, 0, unroll=4)

            def per_win(w, c2):
                row0 = bb * s + w * _W
                pltpu.sync_copy(
                    x_hbm.at[pl.ds(row0, _W), pl.ds(col0, 128)], win)

                def per_strip(sp, c3):
                    c0 = pl.multiple_of(sp * _L, _L)
                    for k in range(1, 8):
                        for dexp in range(k - 1, -1, -1):
                            def st(p, c4, dexp=dexp, k=k):
                                _cexe(win, 0, c0, p, dexp, k)
                                return c4

                            lax.fori_loop(0, _W // 2, st, 0, unroll=4)
                    _merge_128(win, 0, 128, 0, c0)
                    _merge_128(win, 256, 384, 128, c0)
                    _merge_128(win, 0, 128, 0, c0)

                    def hc(i, c4):
                        m = jnp.maximum(acc[i, pl.ds(c0, _L)],
                                        win[127 - i, pl.ds(c0, _L)])
                        acc[i, pl.ds(c0, _L)] = m
                        return c4

                    lax.fori_loop(0, 128, hc, 0, unroll=4)
                    _clean128(acc, 0, c0)
                    return c3

                lax.fori_loop(0, 8, per_strip, 0)
                return c2

            lax.fori_loop(0, nwin, per_win, 0)
            pltpu.sync_copy(
                acc, o_hbm.at[pl.ds(bb * _K, _K), pl.ds(col0, 128)])
            return carry

        lax.fori_loop(0, per_worker, per_tile, 0)

    out2 = sc(x2d)
    return out2.reshape(b, _K, d)


# speedup vs baseline: 8.3474x; 8.3474x over previous
import functools

import jax
import jax.numpy as jnp
from jax import lax
from jax.experimental import pallas as pl
from jax.experimental.pallas import tpu as pltpu

_K = 128
_LANES = 128
_G = 1024


def _ce_sort(v, k, d, row0=0):
    n, lanes = v.shape
    dd = 8 * d
    r_pairs = n // (2 * dd)
    x = v.reshape(r_pairs, 2, dd, lanes)
    a, b = x[:, 0], x[:, 1]
    mx = jnp.maximum(a, b)
    mn = jnp.minimum(a, b)
    nblk = n >> (k + 4)
    if nblk == 0:
        if (row0 >> (k + 3)) & 1:
            top, bot = mn, mx
        else:
            top, bot = mx, mn
    else:
        p = (1 << (k - 1)) // d
        mx5 = mx.reshape(nblk, 2, p, dd, lanes)
        mn5 = mn.reshape(nblk, 2, p, dd, lanes)
        top = jnp.concatenate([mx5[:, 0:1], mn5[:, 1:2]], axis=1)
        bot = jnp.concatenate([mn5[:, 0:1], mx5[:, 1:2]], axis=1)
        top = top.reshape(r_pairs, dd, lanes)
        bot = bot.reshape(r_pairs, dd, lanes)
    return jnp.stack([top, bot], axis=1).reshape(n, lanes)


def _ce_clean(v, d):
    n, lanes = v.shape
    dd = 8 * d
    r_pairs = n // (2 * dd)
    x = v.reshape(r_pairs, 2, dd, lanes)
    a, b = x[:, 0], x[:, 1]
    mx = jnp.maximum(a, b)
    mn = jnp.minimum(a, b)
    return jnp.stack([mx, mn], axis=1).reshape(n, lanes)


def _sort_group(v):
    segs = []
    for si in range(v.shape[0] // 128):
        w = v[si * 128:(si + 1) * 128]
        for k in range(1, 5):
            d = 1 << (k - 1)
            while d:
                w = _ce_sort(w, k, d, row0=si * 128)
                d >>= 1
        segs.append(w)
    v = jnp.concatenate(segs, axis=0)
    for k in range(5, 8):
        d = 1 << (k - 1)
        while d >= 16:
            v = _ce_sort(v, k, d)
            d >>= 1
        segs = []
        for si in range(v.shape[0] // 128):
            w = v[si * 128:(si + 1) * 128]
            ds_ = 8
            while ds_:
                w = _ce_sort(w, k, ds_, row0=si * 128)
                ds_ >>= 1
            segs.append(w)
        v = jnp.concatenate(segs, axis=0)
    return v


def _cleanup(v):
    for d in (64, 32, 16):
        v = _ce_clean(v, d)
    segs = []
    for si in range(v.shape[0] // 128):
        w = v[si * 128:(si + 1) * 128]
        for d in (8, 4, 2, 1):
            w = _ce_clean(w, d)
        segs.append(w)
    return jnp.concatenate(segs, axis=0)


def _blockrev(v):
    n = v.shape[0]
    return jnp.concatenate(
        [v[i * 8:(i + 1) * 8] for i in reversed(range(n // 8))], axis=0)


def _body(x_ref, o_ref, s_ref, *, s):
    ngroups = s // _G

    def p1(g, carry):
        base = pl.multiple_of(g * _G, _G)
        v = x_ref[0, pl.ds(base, _G), :]
        s_ref[pl.ds(base, _G), :] = _sort_group(v)
        return carry

    lax.fori_loop(0, ngroups, p1, 0)

    def merge_groups(ga, gb):
        a = s_ref[pl.ds(pl.multiple_of(ga * _G, _G), _G), :]
        b = s_ref[pl.ds(pl.multiple_of(gb * _G, _G), _G), :]
        m = _cleanup(jnp.maximum(a, _blockrev(b)))
        s_ref[pl.ds(pl.multiple_of(ga * _G, _G), _G), :] = m

    def l1(u, carry):
        merge_groups(2 * u, 2 * u + 1)
        return carry

    lax.fori_loop(0, ngroups // 2, l1, 0)
    merge_groups(0, 2)
    merge_groups(4, 6)
    merge_groups(0, 4)

    v = s_ref[0:_G, :]
    for shift in (1, 2, 4):
        w = _blockrev(v)
        w = jnp.concatenate([w[shift:], w[:shift]], axis=0)
        v = _cleanup(jnp.maximum(v, w))

    o_ref[0] = v.reshape(_K, 8, v.shape[-1])[:, 0, :]


def kernel(inputs):
    b, s, d = inputs.shape
    grid = (b, d // _LANES)
    out = pl.pallas_call(
        functools.partial(_body, s=s),
        grid=grid,
        in_specs=[pl.BlockSpec((1, s, _LANES), lambda bi, j: (bi, 0, j))],
        out_specs=pl.BlockSpec((1, _K, _LANES), lambda bi, j: (bi, 0, j)),
        out_shape=jax.ShapeDtypeStruct((b, _K, d), jnp.float32),
        scratch_shapes=[pltpu.VMEM((s, _LANES), jnp.float32)],
    )(inputs)
    return out
